# Initial kernel scaffold; baseline (speedup 1.0000x reference)
import functools

import jax
import jax.numpy as jnp
from jax import lax
from jax.experimental import pallas as pl
from jax.experimental.pallas import tpu as pltpu
from jax.experimental.pallas import tpu_sc as plsc

B, N, E = 64, 512, 8192
D, L = 128, 5
NODE_FEAT, NUM_EDGE_TYPES, HAND, NC_OUT, FUSION = 34, 8, 193, 9, 256

HALF = D // 2
NWORKERS = 32
GRAPHS_PER_WORKER = B // NWORKERS
EU = 8
VB = 16


def _bn(x, g, b, eps=1e-5):
    m = jnp.mean(x, axis=0, keepdims=True)
    v = jnp.var(x, axis=0, keepdims=True)
    return (x - m) / jnp.sqrt(v + eps) * g + b


def _ln(x, g, b, eps=1e-5):
    m = jnp.mean(x, axis=-1, keepdims=True)
    v = jnp.var(x, axis=-1, keepdims=True)
    return (x - m) / jnp.sqrt(v + eps) * g + b


def _make_sc_agg():
    mesh = plsc.VectorSubcoreMesh(core_axis_name="c", subcore_axis_name="s")

    @functools.partial(
        pl.kernel,
        mesh=mesh,
        out_type=jax.ShapeDtypeStruct((B, 2, N, HALF), jnp.float32),
        scratch_types=[
            pltpu.VMEM((N, HALF), jnp.float32),
            pltpu.VMEM((N, HALF), jnp.float32),
            pltpu.VMEM((E,), jnp.int32),
            pltpu.VMEM((E,), jnp.int32),
            pltpu.VMEM((E,), jnp.int32),
            pltpu.VMEM((NUM_EDGE_TYPES, HALF), jnp.float32),
        ],
    )
    def sc_agg(h2_hbm, src_hbm, dst_hbm, et_hbm, emb2_hbm, out_hbm,
               h_v, agg_v, src_v, dst_v, et_v, emb_v):
        wid = lax.axis_index("s") * 2 + lax.axis_index("c")

        zeros16 = jnp.zeros((VB,), jnp.float32)

        def zero_body(i, _):
            for j in range(HALF // VB):
                agg_v[i, pl.ds(j * VB, VB)] = zeros16
            return 0

        def edge_body(i, _):
            for u in range(EU):
                e = i * EU + u
                s = src_v[e]
                d = dst_v[e]
                t = et_v[e]
                for j in range(HALF // VB):
                    sl = pl.ds(j * VB, VB)
                    hv = h_v[s, sl]
                    ev = emb_v[t, sl]
                    m = jnp.maximum(hv + ev, 0.0)
                    agg_v[d, sl] += m
            return 0

        for a in range(GRAPHS_PER_WORKER):
            g = wid * GRAPHS_PER_WORKER + a
            pltpu.sync_copy(src_hbm.at[g], src_v)
            pltpu.sync_copy(dst_hbm.at[g], dst_v)
            pltpu.sync_copy(et_hbm.at[g], et_v)
            for hf in range(2):
                pltpu.sync_copy(h2_hbm.at[g, hf], h_v)
                pltpu.sync_copy(emb2_hbm.at[hf], emb_v)
                lax.fori_loop(0, N, zero_body, 0)
                lax.fori_loop(0, E // EU, edge_body, 0)
                pltpu.sync_copy(agg_v, out_hbm.at[g, hf])

    return sc_agg


_sc_agg = _make_sc_agg()


def kernel(node_features, edge_index, edge_type, node_mask, handcrafted_features, params):
    p = params
    h = node_features.reshape(-1, NODE_FEAT) @ p['ne_W'] + p['ne_b']
    h = jax.nn.relu(_bn(h, p['ne_bn_g'], p['ne_bn_b'])).reshape(B, N, D)

    src = jnp.asarray(edge_index[:, 0, :], jnp.int32)
    dst = jnp.asarray(edge_index[:, 1, :], jnp.int32)
    et = jnp.asarray(edge_type, jnp.int32)
    emb2 = p['edge_emb'].reshape(NUM_EDGE_TYPES, 2, HALF).transpose(1, 0, 2)

    vn = jnp.broadcast_to(p['vn_init'][None, :], (B, D))
    nm = node_mask[:, :, None]
    outs = [h]
    for l in range(L):
        h2 = h.reshape(B, N, 2, HALF).transpose(0, 2, 1, 3)
        agg2 = _sc_agg(h2, src, dst, et, emb2)
        agg = agg2.transpose(0, 2, 1, 3).reshape(B, N, D)

        h_new = (1.0 + p['eps'][l]) * h + agg
        hf = h_new.reshape(-1, D) @ p['mlp1_W'][l] + p['mlp1_b'][l]
        hf = jax.nn.relu(_bn(hf, p['mlp_bn_g'][l], p['mlp_bn_b'][l]))
        hf = hf @ p['mlp2_W'][l] + p['mlp2_b'][l]
        hf = _bn(hf, p['gbn_g'][l], p['gbn_b'][l])
        h_new = hf.reshape(B, N, D) * nm
        h = _ln(h + h_new, p['ln_g'][l], p['ln_b'][l])
        ns = (h * nm).sum(axis=1)
        vt = vn + ns
        vt = vt @ p['vn1_W'][l] + p['vn1_b'][l]
        vt = jax.nn.relu(_bn(vt, p['vnbn1_g'][l], p['vnbn1_b'][l]))
        vt = vt @ p['vn2_W'][l] + p['vn2_b'][l]
        vt = _bn(vt, p['vnbn2_g'][l], p['vnbn2_b'][l])
        vn_new = vt + vn
        gate = jax.nn.sigmoid(p['vn_gate'][l])
        h = (h + gate * vn_new[:, None, :]) * nm
        vn = vn_new
        outs.append(h)

    h_jk = jnp.concatenate(outs, axis=-1) * nm
    g = h_jk.sum(axis=1)
    g = jax.nn.relu(_bn(g @ p['gp_W'] + p['gp_b'], p['gp_bn_g'], p['gp_bn_b']))
    f = jax.nn.relu(_bn(handcrafted_features @ p['fe1_W'] + p['fe1_b'], p['fe_bn1_g'], p['fe_bn1_b']))
    f = jax.nn.relu(_bn(f @ p['fe2_W'] + p['fe2_b'], p['fe_bn2_g'], p['fe_bn2_b']))
    c = jnp.concatenate([g, f], axis=-1)
    c1 = jax.nn.relu(_bn(c @ p['cl1_W'] + p['cl1_b'], p['cl_bn_g'], p['cl_bn_b']))
    return c1 @ p['cl2_W'] + p['cl2_b']


# speedup vs baseline: 7.8882x; 7.8882x over previous
import functools

import jax
import jax.numpy as jnp
from jax import lax
from jax.experimental import pallas as pl
from jax.experimental.pallas import tpu as pltpu
from jax.experimental.pallas import tpu_sc as plsc

B, N, E = 64, 512, 8192
D, L = 128, 5
NODE_FEAT, NUM_EDGE_TYPES, HAND, NC_OUT, FUSION = 34, 8, 193, 9, 256

NQ = 4
FQ = D // NQ
NWORKERS = 32
GRAPHS_PER_WORKER = B // NWORKERS
EU = 16
VB = 16


def _bn(x, g, b, eps=1e-5):
    m = jnp.mean(x, axis=0, keepdims=True)
    v = jnp.var(x, axis=0, keepdims=True)
    return (x - m) / jnp.sqrt(v + eps) * g + b


def _ln(x, g, b, eps=1e-5):
    m = jnp.mean(x, axis=-1, keepdims=True)
    v = jnp.var(x, axis=-1, keepdims=True)
    return (x - m) / jnp.sqrt(v + eps) * g + b


def _make_sc_agg():
    mesh = plsc.VectorSubcoreMesh(core_axis_name="c", subcore_axis_name="s")

    @functools.partial(
        pl.kernel,
        mesh=mesh,
        compiler_params=pltpu.CompilerParams(use_tc_tiling_on_sc=False),
        out_type=jax.ShapeDtypeStruct((B, NQ, N, FQ), jnp.float32),
        scratch_types=[
            pltpu.VMEM((N, FQ), jnp.float32),
            pltpu.VMEM((N, FQ), jnp.float32),
            pltpu.VMEM((E,), jnp.int32),
            pltpu.VMEM((NUM_EDGE_TYPES, FQ), jnp.float32),
        ],
    )
    def sc_agg(h2_hbm, eidx_hbm, emb2_hbm, out_hbm,
               h_v, agg_v, eidx_v, emb_v):
        wid = lax.axis_index("s") * 2 + lax.axis_index("c")

        zeros16 = jnp.zeros((VB,), jnp.float32)

        def zero_body(i, _):
            for j in range(FQ // VB):
                agg_v[i, pl.ds(j * VB, VB)] = zeros16
            return 0

        def edge_body(i, _):
            packed = eidx_v[pl.ds(i * EU, VB)]
            src16 = packed & 511
            dst16 = (packed >> 9) & 511
            et16 = packed >> 18
            for u in range(EU):
                s = src16[u]
                d = dst16[u]
                t = et16[u]
                for j in range(FQ // VB):
                    sl = pl.ds(j * VB, VB)
                    hv = h_v[s, sl]
                    ev = emb_v[t, sl]
                    m = jnp.maximum(hv + ev, 0.0)
                    agg_v[d, sl] += m
            return 0

        for a in range(GRAPHS_PER_WORKER):
            g = wid * GRAPHS_PER_WORKER + a
            pltpu.sync_copy(eidx_hbm.at[g], eidx_v)
            for hf in range(NQ):
                pltpu.sync_copy(h2_hbm.at[g, hf], h_v)
                pltpu.sync_copy(emb2_hbm.at[hf], emb_v)
                lax.fori_loop(0, N, zero_body, 0)
                lax.fori_loop(0, E // EU, edge_body, 0)
                pltpu.sync_copy(agg_v, out_hbm.at[g, hf])

    return sc_agg


_sc_agg = _make_sc_agg()


def kernel(node_features, edge_index, edge_type, node_mask, handcrafted_features, params):
    p = params
    h = node_features.reshape(-1, NODE_FEAT) @ p['ne_W'] + p['ne_b']
    h = jax.nn.relu(_bn(h, p['ne_bn_g'], p['ne_bn_b'])).reshape(B, N, D)

    src = jnp.asarray(edge_index[:, 0, :], jnp.int32)
    dst = jnp.asarray(edge_index[:, 1, :], jnp.int32)
    et = jnp.asarray(edge_type, jnp.int32)
    eidx = src | (dst << 9) | (et << 18)
    emb2 = p['edge_emb'].reshape(NUM_EDGE_TYPES, NQ, FQ).transpose(1, 0, 2)

    vn = jnp.broadcast_to(p['vn_init'][None, :], (B, D))
    nm = node_mask[:, :, None]
    outs = [h]
    for l in range(L):
        h2 = h.reshape(B, N, NQ, FQ).transpose(0, 2, 1, 3)
        agg2 = _sc_agg(h2, eidx, emb2)
        agg = agg2.transpose(0, 2, 1, 3).reshape(B, N, D)

        h_new = (1.0 + p['eps'][l]) * h + agg
        hf = h_new.reshape(-1, D) @ p['mlp1_W'][l] + p['mlp1_b'][l]
        hf = jax.nn.relu(_bn(hf, p['mlp_bn_g'][l], p['mlp_bn_b'][l]))
        hf = hf @ p['mlp2_W'][l] + p['mlp2_b'][l]
        hf = _bn(hf, p['gbn_g'][l], p['gbn_b'][l])
        h_new = hf.reshape(B, N, D) * nm
        h = _ln(h + h_new, p['ln_g'][l], p['ln_b'][l])
        ns = (h * nm).sum(axis=1)
        vt = vn + ns
        vt = vt @ p['vn1_W'][l] + p['vn1_b'][l]
        vt = jax.nn.relu(_bn(vt, p['vnbn1_g'][l], p['vnbn1_b'][l]))
        vt = vt @ p['vn2_W'][l] + p['vn2_b'][l]
        vt = _bn(vt, p['vnbn2_g'][l], p['vnbn2_b'][l])
        vn_new = vt + vn
        gate = jax.nn.sigmoid(p['vn_gate'][l])
        h = (h + gate * vn_new[:, None, :]) * nm
        vn = vn_new
        outs.append(h)

    h_jk = jnp.concatenate(outs, axis=-1) * nm
    g = h_jk.sum(axis=1)
    g = jax.nn.relu(_bn(g @ p['gp_W'] + p['gp_b'], p['gp_bn_g'], p['gp_bn_b']))
    f = jax.nn.relu(_bn(handcrafted_features @ p['fe1_W'] + p['fe1_b'], p['fe_bn1_g'], p['fe_bn1_b']))
    f = jax.nn.relu(_bn(f @ p['fe2_W'] + p['fe2_b'], p['fe_bn2_g'], p['fe_bn2_b']))
    c = jnp.concatenate([g, f], axis=-1)
    c1 = jax.nn.relu(_bn(c @ p['cl1_W'] + p['cl1_b'], p['cl_bn_g'], p['cl_bn_b']))
    return c1 @ p['cl2_W'] + p['cl2_b']
